# Initial kernel scaffold; baseline (speedup 1.0000x reference)
import jax
import jax.numpy as jnp
from jax.experimental import pallas as pl


def kernel(x, lap_k, sx_k, sy_k):
    raise NotImplementedError("write your pallas kernel here")


# speedup vs baseline: 176.5769x; 176.5769x over previous
import numpy as np
import jax
import jax.numpy as jnp
from jax.experimental import pallas as pl
from jax.experimental.pallas import tpu as pltpu

_NB = 9
_H = 512
_W = 512
_TANS = tuple(float(np.tan(np.radians(-70.0 + 20.0 * k))) for k in range(8))


def _phog_image_kernel(x_ref, out_ref):
    img = x_ref[0]
    H, W = img.shape
    zrow = jnp.zeros((1, W), jnp.float32)
    zcol = jnp.zeros((H, 1), jnp.float32)

    def sd(a):
        return jnp.concatenate([zrow, a[:-1, :]], axis=0)

    def su(a):
        return jnp.concatenate([a[1:, :], zrow], axis=0)

    def sr(a):
        return jnp.concatenate([zcol, a[:, :-1]], axis=1)

    def sl(a):
        return jnp.concatenate([a[:, 1:], zcol], axis=1)

    lap = sd(img) + su(img) + sr(img) + sl(img) - 4.0 * img
    a = sl(lap)
    b = sr(lap)
    m = a - b
    n = a + b + 2.0 * lap
    gx = sd(m) + 2.0 * m + su(m)
    gy = su(n) - sd(n)

    mag = jnp.sqrt(gx * gx + gy * gy + 1e-8)
    flip = (gx < 0.0) | ((gx == 0.0) & (gy < 0.0))
    gxc = jnp.where(flip, -gx, gx)
    gyc = jnp.where(flip, -gy, gy)
    gx_pos = gxc > 0.0
    gx_zero_gy_zero = jnp.logical_not(gx_pos) & (gyc == 0.0)

    def cellrows(v):
        return v.reshape(4, 128, W).sum(axis=1)

    rows = [cellrows(mag)]
    for k, t in enumerate(_TANS):
        ge = gyc >= t * gxc
        if k < 4:
            cond = (gx_pos & ge) | gx_zero_gy_zero
        else:
            cond = gx_pos & ge
        rows.append(cellrows(jnp.where(cond, mag, 0.0)))
    c36 = jnp.concatenate(rows, axis=0)
    cols = [
        jnp.sum(c36[:, c * 128:(c + 1) * 128], axis=1, keepdims=True)
        for c in range(4)
    ]
    m36 = jnp.concatenate(cols, axis=1)

    v = [m36[4 * j:4 * j + 4, :] for j in range(9)]
    d = [v[j] - v[j + 1] for j in range(8)] + [v[8]]

    rows21 = []
    for j in range(_NB):
        dj = d[j]
        r0, r1, r2, r3 = (dj[i:i + 1, :] for i in range(4))
        lvl2 = jnp.concatenate([r0, r1, r2, r3], axis=1)
        u0 = r0 + r1
        u1 = r2 + r3
        lvl1 = jnp.concatenate(
            [u0[:, 0:1] + u0[:, 1:2], u0[:, 2:3] + u0[:, 3:4],
             u1[:, 0:1] + u1[:, 1:2], u1[:, 2:3] + u1[:, 3:4]], axis=1)
        lvl0 = (lvl1[:, 0:1] + lvl1[:, 1:2] + lvl1[:, 2:3] + lvl1[:, 3:4])
        rows21.append(jnp.concatenate([lvl0, lvl1, lvl2], axis=1))
    hc = jnp.concatenate(rows21, axis=0)

    s = jnp.sum(hc, axis=0, keepdims=True)
    h1 = hc / (s + 1e-8)
    nrm = jnp.sqrt(jnp.sum(h1 * h1, axis=0, keepdims=True))
    out_ref[0] = h1 / jnp.maximum(nrm, 1e-12)


def kernel(x, lap_k, sx_k, sy_k):
    B, C, H, W = x.shape
    bc = B * C
    xr = x.reshape(bc, H, W)
    hist = pl.pallas_call(
        _phog_image_kernel,
        grid=(bc,),
        in_specs=[pl.BlockSpec((1, H, W), lambda i: (i, 0, 0))],
        out_specs=pl.BlockSpec((1, _NB, 21), lambda i: (i, 0, 0)),
        out_shape=jax.ShapeDtypeStruct((bc, _NB, 21), jnp.float32),
    )(xr)
    perm = tuple((b + 4) % _NB for b in range(_NB))
    t = hist.reshape(B, C, _NB, 21)[:, :, perm, :]

    def fix(u):
        return u.transpose(0, 1, 3, 2).reshape(B, -1, _NB)

    out = jnp.concatenate(
        [fix(t[..., 0:1]), fix(t[..., 1:5]), fix(t[..., 5:21])], axis=1)
    return out[:, None, :, :]
